# Initial kernel scaffold; baseline (speedup 1.0000x reference)
import jax
import jax.numpy as jnp
from jax.experimental import pallas as pl


def kernel(x, table, pe):
    raise NotImplementedError("write your pallas kernel here")


# speedup vs baseline: 2.0001x; 2.0001x over previous
import functools

import jax
import jax.numpy as jnp
from jax import lax
from jax.experimental import pallas as pl
from jax.experimental.pallas import tpu as pltpu
from jax.experimental.pallas import tpu_sc as plsc

_L = 16


@functools.lru_cache(maxsize=None)
def _build(batch, seq, vocab, d):
    info = plsc.get_sparse_core_info()
    nw = info.num_cores * info.num_subcores
    total = batch * seq
    rows_w = total // nw
    chunk = 128
    n_chunks = rows_w // chunk
    assert rows_w % chunk == 0 and total % nw == 0 and rows_w % seq == 0

    mesh = plsc.VectorSubcoreMesh(core_axis_name="c", subcore_axis_name="s")

    @functools.partial(
        pl.kernel,
        out_type=jax.ShapeDtypeStruct((total, d), jnp.float32),
        mesh=mesh,
        scratch_types=[
            pltpu.VMEM((seq + chunk, d), jnp.float32),
            pltpu.VMEM((chunk,), jnp.int32),
            pltpu.VMEM((chunk, d), jnp.float32),
            pltpu.SemaphoreType.DMA,
        ],
    )
    def k(x_hbm, table_hbm, pe_hbm, out_hbm, pe_v, idx_v, rows_v, sem):
        wid = lax.axis_index("s") * info.num_cores + lax.axis_index("c")
        base_w = wid * rows_w

        pltpu.sync_copy(pe_hbm, pe_v.at[pl.ds(0, seq)])
        pltpu.sync_copy(pe_hbm.at[pl.ds(0, chunk)], pe_v.at[pl.ds(seq, chunk)])

        def chunk_body(c, carry):
            base = base_w + c * chunk
            pe_off = lax.rem(base, seq)
            pltpu.sync_copy(x_hbm.at[pl.ds(base, chunk)], idx_v)
            pltpu.async_copy(table_hbm.at[idx_v], rows_v, sem).wait()

            def row_body(i, carry2):
                for j in range(d // _L):
                    sl = pl.ds(j * _L, _L)
                    rows_v[i, sl] = rows_v[i, sl] + pe_v[pe_off + i, sl]
                return carry2

            lax.fori_loop(0, chunk, row_body, 0, unroll=2)
            pltpu.sync_copy(rows_v, out_hbm.at[pl.ds(base, chunk)])
            return carry

        lax.fori_loop(0, n_chunks, chunk_body, 0)

    return k


def kernel(x, table, pe):
    batch, seq = x.shape
    vocab, d = table.shape
    x_flat = x.reshape(-1).astype(jnp.int32)
    pe2 = pe.reshape(pe.shape[-2], pe.shape[-1])[:seq].astype(jnp.float32)
    out = _build(batch, seq, vocab, d)(x_flat, table, pe2)
    return out.reshape(batch, seq, d)


# speedup vs baseline: 2.6146x; 1.3072x over previous
import functools

import jax
import jax.numpy as jnp
from jax import lax
from jax.experimental import pallas as pl
from jax.experimental.pallas import tpu as pltpu
from jax.experimental.pallas import tpu_sc as plsc

_L = 16
_CH = 128
_NG = 2
_NB = 2


@functools.lru_cache(maxsize=None)
def _build(batch, seq, vocab, d):
    info = plsc.get_sparse_core_info()
    nw = info.num_cores * info.num_subcores
    total = batch * seq
    rows_w = total // nw
    slot = _CH * _NG
    n_steps = rows_w // slot
    assert total % nw == 0 and rows_w % slot == 0 and rows_w % seq == 0
    assert n_steps % _NB == 0 and n_steps >= 2 * _NB

    mesh = plsc.VectorSubcoreMesh(core_axis_name="c", subcore_axis_name="s")

    @functools.partial(
        pl.kernel,
        out_type=jax.ShapeDtypeStruct((total, d), jnp.float32),
        mesh=mesh,
        scratch_types=[
            pltpu.VMEM((seq + _CH, d), jnp.float32),
            pltpu.VMEM((_NG, _CH), jnp.int32),
            pltpu.VMEM((_NG, _CH), jnp.int32),
            pltpu.VMEM((slot, d), jnp.float32),
            pltpu.VMEM((slot, d), jnp.float32),
            pltpu.SemaphoreType.DMA,
            pltpu.SemaphoreType.DMA,
            pltpu.SemaphoreType.DMA,
            pltpu.SemaphoreType.DMA,
        ],
    )
    def k(x_hbm, table_hbm, pe_hbm, out_hbm,
          pe_v, idx0, idx1, rows0, rows1, gsem0, gsem1, ssem0, ssem1):
        wid = lax.axis_index("s") * info.num_cores + lax.axis_index("c")
        base_w = wid * rows_w
        slots = ((idx0, rows0, gsem0, ssem0), (idx1, rows1, gsem1, ssem1))

        pltpu.sync_copy(pe_hbm, pe_v.at[pl.ds(0, seq)])
        pltpu.sync_copy(pe_hbm.at[pl.ds(0, _CH)], pe_v.at[pl.ds(seq, _CH)])

        def stage_idx(g, idx):
            base = base_w + g * slot
            for j in range(_NG):
                pltpu.sync_copy(x_hbm.at[pl.ds(base + j * _CH, _CH)],
                                idx.at[j])

        def fire_gathers(idx, rows, gsem):
            for j in range(_NG):
                pltpu.async_copy(table_hbm.at[idx.at[j]],
                                 rows.at[pl.ds(j * _CH, _CH)], gsem)

        def drain_gathers(idx, rows, gsem):
            for j in range(_NG):
                pltpu.make_async_copy(table_hbm.at[idx.at[j]],
                                      rows.at[pl.ds(j * _CH, _CH)],
                                      gsem).wait()

        def add_pe(g, rows):
            base = base_w + g * slot
            for j in range(_NG):
                pe_off = lax.rem(base + j * _CH, seq)

                def row_body(i, carry):
                    for kk in range(d // _L):
                        sl = pl.ds(kk * _L, _L)
                        rows[j * _CH + i, sl] = (
                            rows[j * _CH + i, sl] + pe_v[pe_off + i, sl])
                    return carry

                lax.fori_loop(0, _CH, row_body, 0, unroll=2)

        for b in range(_NB):
            idx, rows, gsem, _ = slots[b]
            stage_idx(b, idx)
            fire_gathers(idx, rows, gsem)

        def step_body(gg, carry):
            for b in range(_NB):
                g = gg * _NB + b
                idx, rows, gsem, ssem = slots[b]
                drain_gathers(idx, rows, gsem)
                add_pe(g, rows)
                base = base_w + g * slot
                sdesc = pltpu.async_copy(rows, out_hbm.at[pl.ds(base, slot)],
                                         ssem)
                stage_idx(g + _NB, idx)
                sdesc.wait()
                fire_gathers(idx, rows, gsem)
            return carry

        lax.fori_loop(0, (n_steps - _NB) // _NB, step_body, 0)

        for b in range(_NB):
            g = n_steps - _NB + b
            idx, rows, gsem, ssem = slots[b]
            drain_gathers(idx, rows, gsem)
            add_pe(g, rows)
            base = base_w + g * slot
            pltpu.sync_copy(rows, out_hbm.at[pl.ds(base, slot)])

    return k


def kernel(x, table, pe):
    batch, seq = x.shape
    vocab, d = table.shape
    x_flat = x.reshape(-1).astype(jnp.int32)
    pe2 = pe.reshape(pe.shape[-2], pe.shape[-1])[:seq].astype(jnp.float32)
    out = _build(batch, seq, vocab, d)(x_flat, table, pe2)
    return out.reshape(batch, seq, d)


# speedup vs baseline: 8.4636x; 3.2371x over previous
import functools

import jax
import jax.numpy as jnp
from jax import lax
from jax.experimental import pallas as pl
from jax.experimental.pallas import tpu as pltpu
from jax.experimental.pallas import tpu_sc as plsc

_L = 16
_CH = 128
_NG = 2
_NB = 2


@functools.lru_cache(maxsize=None)
def _build(batch, seq, vocab, d):
    info = plsc.get_sparse_core_info()
    nw = info.num_cores * info.num_subcores
    total = batch * seq
    rows_w = total // nw
    slot = _CH * _NG
    n_steps = rows_w // slot
    assert total % nw == 0 and rows_w % slot == 0 and rows_w % seq == 0
    assert n_steps % _NB == 0 and n_steps >= 2 * _NB

    mesh = plsc.VectorSubcoreMesh(core_axis_name="c", subcore_axis_name="s")

    @functools.partial(
        pl.kernel,
        out_type=jax.ShapeDtypeStruct((total, d), jnp.float32),
        mesh=mesh,
        scratch_types=[
            pltpu.VMEM_SHARED((seq + _CH, d), jnp.float32),
            pltpu.VMEM((_NG, _CH), jnp.int32),
            pltpu.VMEM((_NG, _CH), jnp.int32),
            pltpu.VMEM((slot, d), jnp.float32),
            pltpu.VMEM((slot, d), jnp.float32),
            pltpu.SemaphoreType.DMA,
            pltpu.SemaphoreType.DMA,
            pltpu.SemaphoreType.DMA,
            pltpu.SemaphoreType.DMA,
        ],
    )
    def k(x_hbm, table_hbm, pe_hbm, out_hbm,
          pe_v, idx0, idx1, rows0, rows1, gsem0, gsem1, ssem0, ssem1):
        wid = lax.axis_index("s") * info.num_cores + lax.axis_index("c")
        base_w = wid * rows_w
        slots = ((idx0, rows0, gsem0, ssem0), (idx1, rows1, gsem1, ssem1))

        @pl.when(lax.axis_index("s") == 0)
        def _fill_pe():
            pltpu.sync_copy(pe_hbm, pe_v.at[pl.ds(0, seq)])
            pltpu.sync_copy(pe_hbm.at[pl.ds(0, _CH)],
                            pe_v.at[pl.ds(seq, _CH)])

        plsc.subcore_barrier()

        def stage_idx(g, idx):
            base = base_w + g * slot
            for j in range(_NG):
                pltpu.sync_copy(x_hbm.at[pl.ds(base + j * _CH, _CH)],
                                idx.at[j])

        def prefill_pe(g, rows):
            base = base_w + g * slot
            for j in range(_NG):
                pe_off = lax.rem(base + j * _CH, seq)
                pltpu.sync_copy(pe_v.at[pl.ds(pe_off, _CH)],
                                rows.at[pl.ds(j * _CH, _CH)])

        def fire_gathers(idx, rows, gsem):
            for j in range(_NG):
                pltpu.async_copy(table_hbm.at[idx.at[j]],
                                 rows.at[pl.ds(j * _CH, _CH)], gsem,
                                 add=True)

        def drain_gathers(idx, rows, gsem):
            for j in range(_NG):
                pltpu.make_async_copy(table_hbm.at[idx.at[j]],
                                      rows.at[pl.ds(j * _CH, _CH)],
                                      gsem).wait()

        for b in range(_NB):
            idx, rows, gsem, _ = slots[b]
            stage_idx(b, idx)
            prefill_pe(b, rows)
            fire_gathers(idx, rows, gsem)

        def step_body(gg, carry):
            for b in range(_NB):
                g = gg * _NB + b
                idx, rows, gsem, ssem = slots[b]
                drain_gathers(idx, rows, gsem)
                base = base_w + g * slot
                sdesc = pltpu.async_copy(rows, out_hbm.at[pl.ds(base, slot)],
                                         ssem)
                stage_idx(g + _NB, idx)
                sdesc.wait()
                prefill_pe(g + _NB, rows)
                fire_gathers(idx, rows, gsem)
            return carry

        lax.fori_loop(0, (n_steps - _NB) // _NB, step_body, 0)

        for b in range(_NB):
            g = n_steps - _NB + b
            idx, rows, gsem, ssem = slots[b]
            drain_gathers(idx, rows, gsem)
            base = base_w + g * slot
            pltpu.sync_copy(rows, out_hbm.at[pl.ds(base, slot)])

    return k


def kernel(x, table, pe):
    batch, seq = x.shape
    vocab, d = table.shape
    x_flat = x.reshape(-1).astype(jnp.int32)
    pe2 = pe.reshape(pe.shape[-2], pe.shape[-1])[:seq].astype(jnp.float32)
    out = _build(batch, seq, vocab, d)(x_flat, table, pe2)
    return out.reshape(batch, seq, d)


# speedup vs baseline: 9.0467x; 1.0689x over previous
import functools

import jax
import jax.numpy as jnp
from jax import lax
from jax.experimental import pallas as pl
from jax.experimental.pallas import tpu as pltpu
from jax.experimental.pallas import tpu_sc as plsc

_CH = 128
_NB = 4


@functools.lru_cache(maxsize=None)
def _build(batch, seq, vocab, d):
    info = plsc.get_sparse_core_info()
    nw = info.num_cores * info.num_subcores
    total = batch * seq
    rows_w = total // nw
    n_steps = rows_w // _CH
    assert total % nw == 0 and rows_w % _CH == 0 and rows_w % seq == 0
    assert n_steps % _NB == 0 and n_steps >= 2 * _NB

    mesh = plsc.VectorSubcoreMesh(core_axis_name="c", subcore_axis_name="s")

    @functools.partial(
        pl.kernel,
        out_type=jax.ShapeDtypeStruct((total, d), jnp.float32),
        mesh=mesh,
        scratch_types=(
            [pltpu.VMEM_SHARED((seq + _CH, d), jnp.float32)]
            + [pltpu.VMEM((_CH,), jnp.int32) for _ in range(_NB)]
            + [pltpu.VMEM((_CH, d), jnp.float32) for _ in range(_NB)]
            + [pltpu.SemaphoreType.DMA for _ in range(3 * _NB)]
        ),
    )
    def k(x_hbm, table_hbm, pe_hbm, out_hbm, pe_v, *rest):
        idx = rest[0:_NB]
        rows = rest[_NB:2 * _NB]
        gsem = rest[2 * _NB:3 * _NB]
        ssem = rest[3 * _NB:4 * _NB]
        psem = rest[4 * _NB:5 * _NB]
        wid = lax.axis_index("s") * info.num_cores + lax.axis_index("c")
        base_w = wid * rows_w

        @pl.when(lax.axis_index("s") == 0)
        def _fill_pe():
            pltpu.sync_copy(pe_hbm, pe_v.at[pl.ds(0, seq)])
            pltpu.sync_copy(pe_hbm.at[pl.ds(0, _CH)],
                            pe_v.at[pl.ds(seq, _CH)])

        plsc.subcore_barrier()

        def stage_idx(g, b):
            pltpu.sync_copy(x_hbm.at[pl.ds(base_w + g * _CH, _CH)], idx[b])

        def fire_prefill(g, b):
            pe_off = lax.rem(base_w + g * _CH, seq)
            pltpu.async_copy(pe_v.at[pl.ds(pe_off, _CH)], rows[b], psem[b])

        def wait_prefill(b):
            pltpu.make_async_copy(pe_v.at[pl.ds(0, _CH)], rows[b],
                                  psem[b]).wait()

        def fire_gather(b):
            pltpu.async_copy(table_hbm.at[idx[b]], rows[b], gsem[b],
                             add=True)

        def drain_gather(b):
            pltpu.make_async_copy(table_hbm.at[idx[b]], rows[b],
                                  gsem[b]).wait()

        def fire_scat(g, b):
            pltpu.async_copy(rows[b], out_hbm.at[pl.ds(base_w + g * _CH,
                                                       _CH)], ssem[b])

        def wait_scat(g, b):
            pltpu.make_async_copy(rows[b],
                                  out_hbm.at[pl.ds(base_w + g * _CH, _CH)],
                                  ssem[b]).wait()

        for g0 in range(_NB - 1):
            stage_idx(g0, g0)
            fire_prefill(g0, g0)
        for g0 in range(_NB - 2):
            wait_prefill(g0)
            fire_gather(g0)

        def step_body(gg, carry):
            for b in range(_NB):
                g = gg * _NB + b
                drain_gather(b)
                fire_scat(g, b)

                b3 = (b + 3) % _NB

                @pl.when(g >= 1)
                def _ws():
                    wait_scat(g - 1, b3)

                @pl.when(g + _NB - 1 < n_steps)
                def _arm():
                    stage_idx(g + _NB - 1, b3)
                    fire_prefill(g + _NB - 1, b3)

                b2 = (b + 2) % _NB

                @pl.when(g + _NB - 2 < n_steps)
                def _launch():
                    wait_prefill(b2)
                    fire_gather(b2)
            return carry

        lax.fori_loop(0, n_steps // _NB, step_body, 0)

        wait_scat(n_steps - 1, (n_steps - 1) % _NB)

    return k


def kernel(x, table, pe):
    batch, seq = x.shape
    vocab, d = table.shape
    x_flat = x.reshape(-1).astype(jnp.int32)
    pe2 = pe.reshape(pe.shape[-2], pe.shape[-1])[:seq].astype(jnp.float32)
    out = _build(batch, seq, vocab, d)(x_flat, table, pe2)
    return out.reshape(batch, seq, d)


# speedup vs baseline: 9.1304x; 1.0093x over previous
import functools

import jax
import jax.numpy as jnp
from jax import lax
from jax.experimental import pallas as pl
from jax.experimental.pallas import tpu as pltpu
from jax.experimental.pallas import tpu_sc as plsc

_CH = 128
_NB = 5


@functools.lru_cache(maxsize=None)
def _build(batch, seq, vocab, d):
    info = plsc.get_sparse_core_info()
    nw = info.num_cores * info.num_subcores
    total = batch * seq
    rows_w = total // nw
    n_steps = rows_w // _CH
    assert total % nw == 0 and rows_w % _CH == 0 and rows_w % seq == 0
    assert n_steps % _NB == 0 and n_steps >= 2 * _NB

    mesh = plsc.VectorSubcoreMesh(core_axis_name="c", subcore_axis_name="s")

    @functools.partial(
        pl.kernel,
        out_type=jax.ShapeDtypeStruct((total, d), jnp.float32),
        mesh=mesh,
        scratch_types=(
            [pltpu.VMEM_SHARED((seq + _CH, d), jnp.float32)]
            + [pltpu.VMEM((_CH,), jnp.int32) for _ in range(_NB)]
            + [pltpu.VMEM((_CH, d), jnp.float32) for _ in range(_NB)]
            + [pltpu.SemaphoreType.DMA for _ in range(3 * _NB)]
        ),
    )
    def k(x_hbm, table_hbm, pe_hbm, out_hbm, pe_v, *rest):
        idx = rest[0:_NB]
        rows = rest[_NB:2 * _NB]
        gsem = rest[2 * _NB:3 * _NB]
        ssem = rest[3 * _NB:4 * _NB]
        psem = rest[4 * _NB:5 * _NB]
        wid = lax.axis_index("s") * info.num_cores + lax.axis_index("c")
        base_w = wid * rows_w

        @pl.when(lax.axis_index("s") == 0)
        def _fill_pe():
            pltpu.sync_copy(pe_hbm, pe_v.at[pl.ds(0, seq)])
            pltpu.sync_copy(pe_hbm.at[pl.ds(0, _CH)],
                            pe_v.at[pl.ds(seq, _CH)])

        plsc.subcore_barrier()

        def stage_idx(g, b):
            pltpu.sync_copy(x_hbm.at[pl.ds(base_w + g * _CH, _CH)], idx[b])

        def fire_prefill(g, b):
            pe_off = lax.rem(base_w + g * _CH, seq)
            pltpu.async_copy(pe_v.at[pl.ds(pe_off, _CH)], rows[b], psem[b])

        def wait_prefill(b):
            pltpu.make_async_copy(pe_v.at[pl.ds(0, _CH)], rows[b],
                                  psem[b]).wait()

        def fire_gather(b):
            pltpu.async_copy(table_hbm.at[idx[b]], rows[b], gsem[b],
                             add=True)

        def drain_gather(b):
            pltpu.make_async_copy(table_hbm.at[idx[b]], rows[b],
                                  gsem[b]).wait()

        def fire_scat(g, b):
            pltpu.async_copy(rows[b], out_hbm.at[pl.ds(base_w + g * _CH,
                                                       _CH)], ssem[b])

        def wait_scat(g, b):
            pltpu.make_async_copy(rows[b],
                                  out_hbm.at[pl.ds(base_w + g * _CH, _CH)],
                                  ssem[b]).wait()

        for g0 in range(_NB - 1):
            stage_idx(g0, g0)
            fire_prefill(g0, g0)
        for g0 in range(_NB - 2):
            wait_prefill(g0)
            fire_gather(g0)

        def step_body(gg, carry):
            for b in range(_NB):
                g = gg * _NB + b
                drain_gather(b)
                fire_scat(g, b)

                b3 = (b + _NB - 1) % _NB

                @pl.when(g >= 1)
                def _ws():
                    wait_scat(g - 1, b3)

                @pl.when(g + _NB - 1 < n_steps)
                def _arm():
                    stage_idx(g + _NB - 1, b3)
                    fire_prefill(g + _NB - 1, b3)

                b2 = (b + _NB - 2) % _NB

                @pl.when(g + _NB - 2 < n_steps)
                def _launch():
                    wait_prefill(b2)
                    fire_gather(b2)
            return carry

        lax.fori_loop(0, n_steps // _NB, step_body, 0)

        wait_scat(n_steps - 1, (n_steps - 1) % _NB)

    return k


def kernel(x, table, pe):
    batch, seq = x.shape
    vocab, d = table.shape
    x_flat = x.reshape(-1).astype(jnp.int32)
    pe2 = pe.reshape(pe.shape[-2], pe.shape[-1])[:seq].astype(jnp.float32)
    out = _build(batch, seq, vocab, d)(x_flat, table, pe2)
    return out.reshape(batch, seq, d)


# speedup vs baseline: 9.1367x; 1.0007x over previous
import functools

import jax
import jax.numpy as jnp
from jax import lax
from jax.experimental import pallas as pl
from jax.experimental.pallas import tpu as pltpu
from jax.experimental.pallas import tpu_sc as plsc

_CH = 128
_NB = 5


@functools.lru_cache(maxsize=None)
def _build(batch, seq, vocab, d):
    info = plsc.get_sparse_core_info()
    nw = info.num_cores * info.num_subcores
    total = batch * seq
    rows_w = total // nw
    n_steps = rows_w // _CH
    assert total % nw == 0 and rows_w % _CH == 0 and rows_w % seq == 0
    assert n_steps % _NB == 0 and n_steps >= 2 * _NB

    mesh = plsc.VectorSubcoreMesh(core_axis_name="c", subcore_axis_name="s")

    @functools.partial(
        pl.kernel,
        out_type=jax.ShapeDtypeStruct((total, d), jnp.float32),
        mesh=mesh,
        scratch_types=(
            [pltpu.VMEM_SHARED((seq + _CH, d), jnp.float32)]
            + [pltpu.VMEM((_CH,), jnp.int32) for _ in range(_NB)]
            + [pltpu.VMEM((_CH, d), jnp.float32) for _ in range(_NB)]
            + [pltpu.SemaphoreType.DMA for _ in range(3 * _NB)]
        ),
    )
    def k(x_hbm, table_hbm, pe_hbm, out_hbm, pe_v, *rest):
        idx = rest[0:_NB]
        rows = rest[_NB:2 * _NB]
        gsem = rest[2 * _NB:3 * _NB]
        ssem = rest[3 * _NB:4 * _NB]
        psem = rest[4 * _NB:5 * _NB]
        wid = lax.axis_index("s") * info.num_cores + lax.axis_index("c")
        base_w = wid * rows_w

        @pl.when(lax.axis_index("s") == 0)
        def _fill_pe():
            pltpu.sync_copy(pe_hbm, pe_v.at[pl.ds(0, seq)])
            pltpu.sync_copy(pe_hbm.at[pl.ds(0, _CH)],
                            pe_v.at[pl.ds(seq, _CH)])

        plsc.subcore_barrier()

        def stage_idx(g, b):
            pltpu.sync_copy(x_hbm.at[pl.ds(base_w + g * _CH, _CH)], idx[b])

        def fire_prefill(g, b):
            pe_off = lax.rem(base_w + g * _CH, seq)
            pltpu.async_copy(pe_v.at[pl.ds(pe_off, _CH)], rows[b], psem[b])

        def wait_prefill(b):
            pltpu.make_async_copy(pe_v.at[pl.ds(0, _CH)], rows[b],
                                  psem[b]).wait()

        def fire_gather(b):
            pltpu.async_copy(table_hbm.at[idx[b]], rows[b], gsem[b],
                             add=True)

        def drain_gather(b):
            pltpu.make_async_copy(table_hbm.at[idx[b]], rows[b],
                                  gsem[b]).wait()

        def fire_scat(g, b):
            pltpu.async_copy(rows[b], out_hbm.at[pl.ds(base_w + g * _CH,
                                                       _CH)], ssem[b])

        def wait_scat(g, b):
            pltpu.make_async_copy(rows[b],
                                  out_hbm.at[pl.ds(base_w + g * _CH, _CH)],
                                  ssem[b]).wait()

        for g0 in range(_NB - 1):
            stage_idx(g0, g0)
            fire_prefill(g0, g0)
        for g0 in range(_NB - 2):
            wait_prefill(g0)
            fire_gather(g0)

        def step_body(gg, carry):
            for b in range(_NB):
                g = gg * _NB + b
                drain_gather(b)
                fire_scat(g, b)

                ba = (b + _NB - 1) % _NB

                @pl.when(g >= 1)
                def _ws():
                    wait_scat(g - 1, ba)

                @pl.when(g + _NB - 1 < n_steps)
                def _arm():
                    stage_idx(g + _NB - 1, ba)
                    fire_prefill(g + _NB - 1, ba)

                bl = (b + _NB - 2) % _NB

                @pl.when(g + _NB - 2 < n_steps)
                def _launch():
                    wait_prefill(bl)
                    fire_gather(bl)
            return carry

        lax.fori_loop(0, n_steps // _NB, step_body, 0)

        wait_scat(n_steps - 1, (n_steps - 1) % _NB)

    return k


def kernel(x, table, pe):
    batch, seq = x.shape
    vocab, d = table.shape
    x_flat = x.reshape(-1).astype(jnp.int32)
    pe2 = pe.reshape(pe.shape[-2], pe.shape[-1])[:seq].astype(jnp.float32)
    out = _build(batch, seq, vocab, d)(x_flat, table, pe2)
    return out.reshape(batch, seq, d)
